# baseline (speedup 1.0000x reference)
import functools

import jax
import jax.numpy as jnp
from jax import lax
from jax.experimental import pallas as pl
from jax.experimental.pallas import tpu as pltpu
from jax.experimental.pallas import tpu_sc as plsc

NC = 2
NS = 16
NW = NC * NS
CHUNK = 128
N_ACC = 10112
ROWS_PER_TILE = N_ACC // NS


NBUF = 3


def _sc_scatter_stage(table, src3, dst3, zeros_acc, n, width, n_chunks,
                      do_gather, ones_rows=None):
  mesh = plsc.VectorSubcoreMesh(core_axis_name="c", subcore_axis_name="s",
                                num_cores=NC, num_subcores=NS)
  assert n_chunks % NBUF == 0
  n_groups = n_chunks // NBUF

  scratch = (
      [
          pltpu.VMEM((n_chunks, CHUNK), jnp.int32),
          pltpu.VMEM((n_chunks, CHUNK), jnp.int32),
          pltpu.VMEM((NBUF, CHUNK, width), jnp.float32),
          pltpu.VMEM_SHARED((N_ACC, width), jnp.float32),
          pltpu.VMEM_SHARED((n, width), jnp.float32),
      ]
      + [pltpu.SemaphoreType.DMA] * (2 * NBUF)
  )

  @functools.partial(
      pl.kernel,
      out_type=jax.ShapeDtypeStruct((NC * N_ACC, width), jnp.float32),
      mesh=mesh,
      scratch_types=scratch,
      compiler_params=pltpu.CompilerParams(use_tc_tiling_on_sc=False),
  )
  def body(*refs):
    (table_h, src_h, dst_h, zeros_h, out_h, src_v, dst_v, rows_v, acc_sh,
     tbl_sh) = refs[:10]
    gsem = refs[10:10 + NBUF]
    ssem = refs[10 + NBUF:10 + 2 * NBUF]
    c = lax.axis_index("c")
    s = lax.axis_index("s")
    wid = s * NC + c

    pltpu.sync_copy(src_h.at[wid], src_v)
    pltpu.sync_copy(dst_h.at[wid], dst_v)

    pltpu.sync_copy(zeros_h.at[pl.ds(s * ROWS_PER_TILE, ROWS_PER_TILE)],
                    acc_sh.at[pl.ds(s * ROWS_PER_TILE, ROWS_PER_TILE)])
    if not do_gather:
      pltpu.sync_copy(table_h, rows_v.at[0])
    else:
      tbl_rows = tbl_sh.shape[0] // NS
      pltpu.sync_copy(table_h.at[pl.ds(s * tbl_rows, tbl_rows)],
                      tbl_sh.at[pl.ds(s * tbl_rows, tbl_rows)])
    plsc.subcore_barrier()

    if do_gather:
      for b in range(NBUF):
        pltpu.async_copy(tbl_sh.at[src_v.at[b]], rows_v.at[b], gsem[b])

      def group(g, carry):
        base = g * NBUF
        for b in range(NBUF):
          j = base + b
          pltpu.make_async_copy(tbl_sh.at[src_v.at[j]], rows_v.at[b],
                                gsem[b]).wait()
          pltpu.async_copy(rows_v.at[b], acc_sh.at[dst_v.at[j]], ssem[b],
                           add=True)
          pltpu.make_async_copy(rows_v.at[b], acc_sh.at[dst_v.at[j]],
                                ssem[b]).wait()
          jn = j + NBUF

          @pl.when(jn < n_chunks)
          def _():
            pltpu.async_copy(tbl_sh.at[src_v.at[jn]], rows_v.at[b], gsem[b])

        return carry

      lax.fori_loop(0, n_groups, group, 0)
    else:
      for b in range(NBUF):
        pltpu.async_copy(rows_v.at[0], acc_sh.at[dst_v.at[b]], ssem[b],
                         add=True)

      def group0(g, carry):
        base = g * NBUF
        for b in range(NBUF):
          j = base + b
          pltpu.make_async_copy(rows_v.at[0], acc_sh.at[dst_v.at[j]],
                                ssem[b]).wait()
          jn = j + NBUF

          @pl.when(jn < n_chunks)
          def _():
            pltpu.async_copy(rows_v.at[0], acc_sh.at[dst_v.at[jn]], ssem[b],
                             add=True)

        return carry

      lax.fori_loop(0, n_groups, group0, 0)

    plsc.subcore_barrier()

    base = s * ROWS_PER_TILE
    pltpu.sync_copy(acc_sh.at[pl.ds(base, ROWS_PER_TILE)],
                    out_h.at[pl.ds(c * N_ACC + base, ROWS_PER_TILE)])

  if do_gather:
    return body(table, src3, dst3, zeros_acc)
  else:
    return body(ones_rows, src3, dst3, zeros_acc)


def _split_partials(out2, n):
  return out2[:n], out2[N_ACC:N_ACC + n]


RB = 2000


def _row_specs(shapes):
  specs = []
  for kind, shp in shapes:
    if kind == "row":
      specs.append(pl.BlockSpec((RB, shp), lambda i: (i, 0)))
    else:
      specs.append(pl.BlockSpec(shp, lambda i, r=len(shp): (0,) * r))
  return specs


def _tc_call(fn, in_shapes, out_shapes, n, args):
  grid = (n // RB,)
  return pl.pallas_call(
      fn,
      grid=grid,
      in_specs=_row_specs(in_shapes),
      out_specs=_row_specs(out_shapes),
      out_shape=[jax.ShapeDtypeStruct((n, k), jnp.float32)
                 for _, k in out_shapes],
  )(*args)


def _tc1a(x, W, t1_o):
  t1_o[...] = jnp.dot(x[...], W[...], preferred_element_type=jnp.float32)


def _tc1b(dA, dB, t1, dinv_o, ys1_o):
  deg = dA[:, :1] + dB[:, :1] + 1.0
  dinv = lax.rsqrt(deg)
  dinv_o[...] = dinv
  ys1_o[...] = dinv * t1[...]


def _tc_mid(relu, t_p, Sa, Sb, dinv, Wn, b, t_o, ys_o, h_o=None):
  dv = dinv[...]
  u = dv * (Sa[...] + Sb[...]) + dv * dv * t_p[...] + b[...]
  if relu:
    u = jnp.maximum(u, 0.0)
  if h_o is not None:
    h_o[...] = u
  t = jnp.dot(u, Wn[...], preferred_element_type=jnp.float32)
  t_o[...] = t
  ys_o[...] = dv * t


def _tc4(t3, Sa, Sb, dinv, b, x1_o, ys4_o):
  dv = dinv[...]
  x1 = jnp.maximum(dv * (Sa[...] + Sb[...]) + dv * dv * t3[...] + b[...], 0.0)
  x1_o[...] = x1
  ys4_o[...] = dv * x1


def _tc_last(Sa, Sb, dinv, t, W, b, o):
  dv = dinv[...]
  A = dv * (Sa[...] + Sb[...]) + dv * dv * t[...]
  o[...] = jnp.dot(A, W[...], preferred_element_type=jnp.float32) + b[...]


MB_R = 2000
MB_C = 2048


def _adj_fused(S5ai, S5bi, dvi, hi, S5aj, S5bj, dvj, hj, Ws1, bs1, o):
  W = Ws1[...]
  b = bs1[...]
  di = dvi[...]
  Ai = di * (S5ai[...] + S5bi[...]) + di * di * hi[...]
  hri = jnp.dot(Ai, W, preferred_element_type=jnp.float32) + b
  dj = dvj[...]
  Aj = dj * (S5aj[...] + S5bj[...]) + dj * dj * hj[...]
  hrj = jnp.dot(Aj, W, preferred_element_type=jnp.float32) + b
  o[...] = jnp.dot(hri.astype(jnp.bfloat16), hrj.astype(jnp.bfloat16).T,
                   preferred_element_type=jnp.float32)


def kernel(x, edge_index, W_e1, b_e1, W_e2, b_e2, W_a1, b_a1, W_a2, b_a2,
           W_s1, b_s1):
  n = x.shape[0]
  e = edge_index.shape[1]

  n_chunks = -(-e // (NW * CHUNK))
  n_chunks = -(-n_chunks // NBUF) * NBUF
  e_pad = n_chunks * NW * CHUNK
  src = jnp.concatenate(
      [edge_index[0], jnp.zeros((e_pad - e,), jnp.int32)])
  dst = jnp.concatenate(
      [edge_index[1], jnp.full((e_pad - e,), n, jnp.int32)])
  src3 = src.reshape(NW, n_chunks, CHUNK)
  dst3 = dst.reshape(NW, n_chunks, CHUNK)

  zeros64 = jnp.zeros((N_ACC, 64), jnp.float32)
  zeros128 = jnp.zeros((N_ACC, 128), jnp.float32)
  zeros16 = jnp.zeros((N_ACC, 16), jnp.float32)
  ones16 = jnp.ones((CHUNK, 16), jnp.float32)

  stage = functools.partial(_sc_scatter_stage, n=n, n_chunks=n_chunks)

  degp = stage(None, src3, dst3, zeros16, width=16, do_gather=False,
               ones_rows=ones16)
  dA, dB = _split_partials(degp, n)

  b_e1r = b_e1.reshape(1, 64)
  b_e2r = b_e2.reshape(1, 64)
  b_a1r = b_a1.reshape(1, 64)
  t1 = _tc_call(
      _tc1a,
      [("row", 128), ("full", (128, 64))],
      [("row", 64)],
      n, (x, W_e1))[0]
  dinv, ys1 = _tc_call(
      _tc1b,
      [("row", 16), ("row", 16), ("row", 64)],
      [("row", 1), ("row", 64)],
      n, (dA, dB, t1))

  s1 = stage(ys1, src3, dst3, zeros64, width=64, do_gather=True)
  S1a, S1b = _split_partials(s1, n)

  t2, ys2 = _tc_call(
      functools.partial(_tc_mid, True),
      [("row", 64), ("row", 64), ("row", 64), ("row", 1),
       ("full", (64, 64)), ("full", (1, 64))],
      [("row", 64), ("row", 64)],
      n, (t1, S1a, S1b, dinv, W_e2, b_e1r))

  s2 = stage(ys2, src3, dst3, zeros64, width=64, do_gather=True)
  S2a, S2b = _split_partials(s2, n)

  def _tc3(t2_r, Sa, Sb, dinv, Wn, b, t_o, ys_o, h_o, ys5_o):
    dv = dinv[...]
    h_v = dv * (Sa[...] + Sb[...]) + dv * dv * t2_r[...] + b[...]
    h_o[...] = h_v
    ys5_o[...] = dv * h_v
    t = jnp.dot(h_v, Wn[...], preferred_element_type=jnp.float32)
    t_o[...] = t
    ys_o[...] = dv * t

  t3, ys3, h, ys5 = _tc_call(
      _tc3,
      [("row", 64), ("row", 64), ("row", 64), ("row", 1),
       ("full", (64, 64)), ("full", (1, 64))],
      [("row", 64), ("row", 64), ("row", 64), ("row", 64)],
      n, (t2, S2a, S2b, dinv, W_a1, b_e2r))

  s5 = stage(ys5, src3, dst3, zeros64, width=64, do_gather=True)
  S5a, S5b = _split_partials(s5, n)

  s3 = stage(ys3, src3, dst3, zeros64, width=64, do_gather=True)
  S3a, S3b = _split_partials(s3, n)

  ri = lambda i, j: (i, 0)
  rj = lambda i, j: (j, 0)
  adj_ = pl.pallas_call(
      _adj_fused,
      grid=(n // MB_R, pl.cdiv(n, MB_C)),
      in_specs=[
          pl.BlockSpec((MB_R, 64), ri), pl.BlockSpec((MB_R, 64), ri),
          pl.BlockSpec((MB_R, 1), ri), pl.BlockSpec((MB_R, 64), ri),
          pl.BlockSpec((MB_C, 64), rj), pl.BlockSpec((MB_C, 64), rj),
          pl.BlockSpec((MB_C, 1), rj), pl.BlockSpec((MB_C, 64), rj),
          pl.BlockSpec((64, 128), lambda i, j: (0, 0)),
          pl.BlockSpec((1, 128), lambda i, j: (0, 0)),
      ],
      out_specs=pl.BlockSpec((MB_R, MB_C), lambda i, j: (i, j)),
      out_shape=jax.ShapeDtypeStruct((n, n), jnp.float32),
  )(S5a, S5b, dinv, h, S5a, S5b, dinv, h, W_s1, b_s1.reshape(1, 128))

  x1, ys4 = _tc_call(
      _tc4,
      [("row", 64), ("row", 64), ("row", 64), ("row", 1),
       ("full", (1, 64))],
      [("row", 64), ("row", 64)],
      n, (t3, S3a, S3b, dinv, b_a1r))

  s4 = stage(ys4, src3, dst3, zeros64, width=64, do_gather=True)
  S4a, S4b = _split_partials(s4, n)

  x_ = _tc_call(
      _tc_last,
      [("row", 64), ("row", 64), ("row", 1), ("row", 64),
       ("full", (64, 128)), ("full", (1, 128))],
      [("row", 128)],
      n, (S4a, S4b, dinv, x1, W_a2, b_a2.reshape(1, 128)))[0]

  return (x_, adj_)


# speedup vs baseline: 1.0342x; 1.0342x over previous
import functools

import jax
import jax.numpy as jnp
from jax import lax
from jax.experimental import pallas as pl
from jax.experimental.pallas import tpu as pltpu
from jax.experimental.pallas import tpu_sc as plsc

NC = 2
NS = 16
NW = NC * NS
CHUNK = 128
N_ACC = 10112
ROWS_PER_TILE = N_ACC // NS


NBUF = 3


def _sc_scatter_stage(table, src3, dst3, zeros_acc, n, width, n_chunks,
                      do_gather, ones_rows=None):
  mesh = plsc.VectorSubcoreMesh(core_axis_name="c", subcore_axis_name="s",
                                num_cores=NC, num_subcores=NS)
  assert n_chunks % NBUF == 0
  n_groups = n_chunks // NBUF

  scratch = (
      [
          pltpu.VMEM((n_chunks, CHUNK), jnp.int32),
          pltpu.VMEM((n_chunks, CHUNK), jnp.int32),
          pltpu.VMEM((NBUF, CHUNK, width), jnp.float32),
          pltpu.VMEM_SHARED((N_ACC, width), jnp.float32),
          pltpu.VMEM_SHARED((n, width), jnp.float32),
      ]
      + [pltpu.SemaphoreType.DMA] * (2 * NBUF)
  )

  @functools.partial(
      pl.kernel,
      out_type=jax.ShapeDtypeStruct((NC * N_ACC, width), jnp.float32),
      mesh=mesh,
      scratch_types=scratch,
      compiler_params=pltpu.CompilerParams(use_tc_tiling_on_sc=False),
  )
  def body(*refs):
    (table_h, src_h, dst_h, zeros_h, out_h, src_v, dst_v, rows_v, acc_sh,
     tbl_sh) = refs[:10]
    gsem = refs[10:10 + NBUF]
    ssem = refs[10 + NBUF:10 + 2 * NBUF]
    c = lax.axis_index("c")
    s = lax.axis_index("s")
    wid = s * NC + c

    pltpu.sync_copy(src_h.at[wid], src_v)
    pltpu.sync_copy(dst_h.at[wid], dst_v)

    pltpu.sync_copy(zeros_h.at[pl.ds(s * ROWS_PER_TILE, ROWS_PER_TILE)],
                    acc_sh.at[pl.ds(s * ROWS_PER_TILE, ROWS_PER_TILE)])
    if not do_gather:
      pltpu.sync_copy(table_h, rows_v.at[0])
    else:
      tbl_rows = tbl_sh.shape[0] // NS
      pltpu.sync_copy(table_h.at[pl.ds(s * tbl_rows, tbl_rows)],
                      tbl_sh.at[pl.ds(s * tbl_rows, tbl_rows)])
    plsc.subcore_barrier()

    if do_gather:
      for b in range(NBUF):
        pltpu.async_copy(tbl_sh.at[src_v.at[b]], rows_v.at[b], gsem[b])

      def group(g, carry):
        base = g * NBUF
        for b in range(NBUF):
          j = base + b
          pltpu.make_async_copy(tbl_sh.at[src_v.at[j]], rows_v.at[b],
                                gsem[b]).wait()
          pltpu.async_copy(rows_v.at[b], acc_sh.at[dst_v.at[j]], ssem[b],
                           add=True)
          pltpu.make_async_copy(rows_v.at[b], acc_sh.at[dst_v.at[j]],
                                ssem[b]).wait()
          jn = j + NBUF

          @pl.when(jn < n_chunks)
          def _():
            pltpu.async_copy(tbl_sh.at[src_v.at[jn]], rows_v.at[b], gsem[b])

        return carry

      lax.fori_loop(0, n_groups, group, 0)
    else:
      for b in range(NBUF):
        pltpu.async_copy(rows_v.at[0], acc_sh.at[dst_v.at[b]], ssem[b],
                         add=True)

      def group0(g, carry):
        base = g * NBUF
        for b in range(NBUF):
          j = base + b
          pltpu.make_async_copy(rows_v.at[0], acc_sh.at[dst_v.at[j]],
                                ssem[b]).wait()
          jn = j + NBUF

          @pl.when(jn < n_chunks)
          def _():
            pltpu.async_copy(rows_v.at[0], acc_sh.at[dst_v.at[jn]], ssem[b],
                             add=True)

        return carry

      lax.fori_loop(0, n_groups, group0, 0)

    plsc.subcore_barrier()

    base = s * ROWS_PER_TILE
    pltpu.sync_copy(acc_sh.at[pl.ds(base, ROWS_PER_TILE)],
                    out_h.at[pl.ds(c * N_ACC + base, ROWS_PER_TILE)])

  if do_gather:
    return body(table, src3, dst3, zeros_acc)
  else:
    return body(ones_rows, src3, dst3, zeros_acc)


def _split_partials(out2, n):
  return out2[:n], out2[N_ACC:N_ACC + n]


RB = 2000


def _row_specs(shapes):
  specs = []
  for kind, shp in shapes:
    if kind == "row":
      specs.append(pl.BlockSpec((RB, shp), lambda i: (i, 0)))
    else:
      specs.append(pl.BlockSpec(shp, lambda i, r=len(shp): (0,) * r))
  return specs


def _tc_call(fn, in_shapes, out_shapes, n, args, out_dtype=jnp.float32):
  grid = (n // RB,)
  return pl.pallas_call(
      fn,
      grid=grid,
      in_specs=_row_specs(in_shapes),
      out_specs=_row_specs(out_shapes),
      out_shape=[jax.ShapeDtypeStruct((n, k), out_dtype)
                 for _, k in out_shapes],
  )(*args)


def _tc1a(x, W, t1_o):
  t1_o[...] = jnp.dot(x[...], W[...], preferred_element_type=jnp.float32)


def _tc1b(dA, dB, t1, dinv_o, ys1_o):
  deg = dA[:, :1] + dB[:, :1] + 1.0
  dinv = lax.rsqrt(deg)
  dinv_o[...] = dinv
  ys1_o[...] = dinv * t1[...]


def _tc_mid(relu, t_p, Sa, Sb, dinv, Wn, b, t_o, ys_o, h_o=None):
  dv = dinv[...]
  u = dv * (Sa[...] + Sb[...]) + dv * dv * t_p[...] + b[...]
  if relu:
    u = jnp.maximum(u, 0.0)
  if h_o is not None:
    h_o[...] = u
  t = jnp.dot(u, Wn[...], preferred_element_type=jnp.float32)
  t_o[...] = t
  ys_o[...] = dv * t


def _tc4(t3, Sa, Sb, dinv, b, x1_o, ys4_o):
  dv = dinv[...]
  x1 = jnp.maximum(dv * (Sa[...] + Sb[...]) + dv * dv * t3[...] + b[...], 0.0)
  x1_o[...] = x1
  ys4_o[...] = dv * x1


def _tc_last(Sa, Sb, dinv, t, W, b, o):
  dv = dinv[...]
  A = dv * (Sa[...] + Sb[...]) + dv * dv * t[...]
  o[...] = jnp.dot(A, W[...], preferred_element_type=jnp.float32) + b[...]


MB_R = 400


def _tc_hb(Sa, Sb, dinv, t, W, b, o):
  dv = dinv[...]
  A = dv * (Sa[...] + Sb[...]) + dv * dv * t[...]
  h_ = jnp.dot(A, W[...], preferred_element_type=jnp.float32) + b[...]
  o[...] = h_.astype(jnp.bfloat16)


def _adj_kernel(hbi, hTb, o):
  o[...] = jnp.dot(hbi[...], hTb[...], preferred_element_type=jnp.float32)


def kernel(x, edge_index, W_e1, b_e1, W_e2, b_e2, W_a1, b_a1, W_a2, b_a2,
           W_s1, b_s1):
  n = x.shape[0]
  e = edge_index.shape[1]

  n_chunks = -(-e // (NW * CHUNK))
  n_chunks = -(-n_chunks // NBUF) * NBUF
  e_pad = n_chunks * NW * CHUNK
  src = jnp.concatenate(
      [edge_index[0], jnp.zeros((e_pad - e,), jnp.int32)])
  dst = jnp.concatenate(
      [edge_index[1], jnp.full((e_pad - e,), n, jnp.int32)])
  src3 = src.reshape(NW, n_chunks, CHUNK)
  dst3 = dst.reshape(NW, n_chunks, CHUNK)

  zeros64 = jnp.zeros((N_ACC, 64), jnp.float32)
  zeros128 = jnp.zeros((N_ACC, 128), jnp.float32)
  zeros16 = jnp.zeros((N_ACC, 16), jnp.float32)
  ones16 = jnp.ones((CHUNK, 16), jnp.float32)

  stage = functools.partial(_sc_scatter_stage, n=n, n_chunks=n_chunks)

  b_e1r = b_e1.reshape(1, 64)
  b_e2r = b_e2.reshape(1, 64)
  b_a1r = b_a1.reshape(1, 64)
  t1 = _tc_call(
      _tc1a,
      [("row", 128), ("full", (128, 64))],
      [("row", 64)],
      n, (x, W_e1))[0]

  degp = stage(None, src3, dst3, zeros16, width=16, do_gather=False,
               ones_rows=ones16)
  dA, dB = _split_partials(degp, n)
  dinv, ys1 = _tc_call(
      _tc1b,
      [("row", 16), ("row", 16), ("row", 64)],
      [("row", 1), ("row", 64)],
      n, (dA, dB, t1))

  s1 = stage(ys1, src3, dst3, zeros64, width=64, do_gather=True)
  S1a, S1b = _split_partials(s1, n)

  t2, ys2 = _tc_call(
      functools.partial(_tc_mid, True),
      [("row", 64), ("row", 64), ("row", 64), ("row", 1),
       ("full", (64, 64)), ("full", (1, 64))],
      [("row", 64), ("row", 64)],
      n, (t1, S1a, S1b, dinv, W_e2, b_e1r))

  s2 = stage(ys2, src3, dst3, zeros64, width=64, do_gather=True)
  S2a, S2b = _split_partials(s2, n)

  def _tc3(t2_r, Sa, Sb, dinv, Wn, b, t_o, ys_o, h_o, ys5_o):
    dv = dinv[...]
    h_v = dv * (Sa[...] + Sb[...]) + dv * dv * t2_r[...] + b[...]
    h_o[...] = h_v
    ys5_o[...] = dv * h_v
    t = jnp.dot(h_v, Wn[...], preferred_element_type=jnp.float32)
    t_o[...] = t
    ys_o[...] = dv * t

  t3, ys3, h, ys5 = _tc_call(
      _tc3,
      [("row", 64), ("row", 64), ("row", 64), ("row", 1),
       ("full", (64, 64)), ("full", (1, 64))],
      [("row", 64), ("row", 64), ("row", 64), ("row", 64)],
      n, (t2, S2a, S2b, dinv, W_a1, b_e2r))

  s5 = stage(ys5, src3, dst3, zeros64, width=64, do_gather=True)
  S5a, S5b = _split_partials(s5, n)

  hb = _tc_call(
      _tc_hb,
      [("row", 64), ("row", 64), ("row", 1), ("row", 64),
       ("full", (64, 128)), ("full", (1, 128))],
      [("row", 128)],
      n, (S5a, S5b, dinv, h, W_s1, b_s1.reshape(1, 128)),
      out_dtype=jnp.bfloat16)[0]
  hTb = hb.T

  s3 = stage(ys3, src3, dst3, zeros64, width=64, do_gather=True)
  S3a, S3b = _split_partials(s3, n)

  adj_ = pl.pallas_call(
      _adj_kernel,
      grid=(n // MB_R,),
      in_specs=[
          pl.BlockSpec((MB_R, 128), lambda i: (i, 0)),
          pl.BlockSpec((128, n), lambda i: (0, 0)),
      ],
      out_specs=pl.BlockSpec((MB_R, n), lambda i: (i, 0)),
      out_shape=jax.ShapeDtypeStruct((n, n), jnp.float32),
  )(hb, hTb)

  x1, ys4 = _tc_call(
      _tc4,
      [("row", 64), ("row", 64), ("row", 64), ("row", 1),
       ("full", (1, 64))],
      [("row", 64), ("row", 64)],
      n, (t3, S3a, S3b, dinv, b_a1r))

  s4 = stage(ys4, src3, dst3, zeros64, width=64, do_gather=True)
  S4a, S4b = _split_partials(s4, n)

  x_ = _tc_call(
      _tc_last,
      [("row", 64), ("row", 64), ("row", 1), ("row", 64),
       ("full", (64, 128)), ("full", (1, 128))],
      [("row", 128)],
      n, (S4a, S4b, dinv, x1, W_a2, b_a2.reshape(1, 128)))[0]

  return (x_, adj_)
